# Initial kernel scaffold; baseline (speedup 1.0000x reference)
import jax
import jax.numpy as jnp
from jax.experimental import pallas as pl


def kernel(x, edge_index):
    raise NotImplementedError("write your pallas kernel here")


# speedup vs baseline: 11.3467x; 11.3467x over previous
import functools

import jax
import jax.numpy as jnp
from jax import lax
from jax.experimental import pallas as pl
from jax.experimental.pallas import tpu as pltpu
from jax.experimental.pallas import tpu_sc as plsc

N_NODES = 10000
D_FEAT = 128
NC, NS = 2, 16
NW = NC * NS
EB = 128
ACC_ROWS = 10240
STRIPE = ACC_ROWS // NS
DUMMY = N_NODES

_MESH = plsc.VectorSubcoreMesh(core_axis_name="c", subcore_axis_name="s",
                               num_cores=NC, num_subcores=NS)


def _degree_body(rows_hbm, ones_hbm, zeros_hbm, out_hbm, ridx, ones_v, acc):
    nb = ridx.shape[0]
    cid = lax.axis_index("c")
    sid = lax.axis_index("s")
    wid = sid * NC + cid
    pltpu.sync_copy(zeros_hbm, acc.at[pl.ds(sid * STRIPE, STRIPE)])
    pltpu.sync_copy(rows_hbm.at[wid], ridx)
    pltpu.sync_copy(ones_hbm, ones_v)
    plsc.subcore_barrier()

    def step(b, carry):
        pltpu.sync_copy(ones_v, acc.at[ridx.at[b]], add=True)
        return carry

    lax.fori_loop(0, nb, step, 0)
    plsc.subcore_barrier()
    pltpu.sync_copy(acc.at[pl.ds(sid * STRIPE, STRIPE)],
                    out_hbm.at[cid, pl.ds(sid * STRIPE, STRIPE)])


def _make_degree(nb):
    return pl.kernel(
        _degree_body,
        out_type=jax.ShapeDtypeStruct((NC, ACC_ROWS, D_FEAT), jnp.float32),
        mesh=_MESH,
        scratch_types=[
            pltpu.VMEM((nb, EB), jnp.int32),
            pltpu.VMEM((EB, D_FEAT), jnp.float32),
            pltpu.VMEM_SHARED((ACC_ROWS, D_FEAT), jnp.float32),
        ],
    )


def _spmm_body(g_hbm, rows_hbm, cols_hbm, zeros_hbm, out_hbm,
               ridx, cidx, buf, acc, sem):
    nb = ridx.shape[0]
    cid = lax.axis_index("c")
    sid = lax.axis_index("s")
    wid = sid * NC + cid
    pltpu.sync_copy(zeros_hbm, acc.at[pl.ds(sid * STRIPE, STRIPE)])
    pltpu.sync_copy(rows_hbm.at[wid], ridx)
    pltpu.sync_copy(cols_hbm.at[wid], cidx)
    plsc.subcore_barrier()

    def step(b, carry):
        pltpu.async_copy(g_hbm.at[cidx.at[b]], buf, sem).wait()
        pltpu.sync_copy(buf, acc.at[ridx.at[b]], add=True)
        return carry

    lax.fori_loop(0, nb, step, 0)
    plsc.subcore_barrier()
    pltpu.sync_copy(acc.at[pl.ds(sid * STRIPE, STRIPE)],
                    out_hbm.at[cid, pl.ds(sid * STRIPE, STRIPE)])


def _make_spmm(nb):
    return pl.kernel(
        _spmm_body,
        out_type=jax.ShapeDtypeStruct((NC, ACC_ROWS, D_FEAT), jnp.float32),
        mesh=_MESH,
        scratch_types=[
            pltpu.VMEM((nb, EB), jnp.int32),
            pltpu.VMEM((nb, EB), jnp.int32),
            pltpu.VMEM((EB, D_FEAT), jnp.float32),
            pltpu.VMEM_SHARED((ACC_ROWS, D_FEAT), jnp.float32),
            pltpu.SemaphoreType.DMA,
        ],
    )


def _prep_body(x_ref, degp_ref, g0_ref):
    x = x_ref[...]
    n = x.shape[0]
    mean = jnp.mean(x, axis=0, keepdims=True)
    xc = x - mean
    var = jnp.sum(xc * xc, axis=0, keepdims=True) / (n - 1)
    rstd = jnp.where(var > 0.0, lax.rsqrt(var), 1.0)
    deg = degp_ref[0, :N_NODES, 0:1] + degp_ref[1, :N_NODES, 0:1] + 1.0
    s = lax.rsqrt(deg)
    g0_ref[...] = xc * rstd * s


def _combine_body(zp_ref, g_ref, degp_ref, out_ref, *, last_hop):
    deg = degp_ref[0, :N_NODES, 0:1] + degp_ref[1, :N_NODES, 0:1] + 1.0
    scale = lax.rsqrt(deg) if last_hop else 1.0 / deg
    z = zp_ref[0, :N_NODES, :] + zp_ref[1, :N_NODES, :] + g_ref[...]
    out_ref[...] = z * scale


def kernel(x, edge_index):
    e = edge_index.shape[1]
    nb = -(-e // (NW * EB))
    epad = NW * nb * EB
    row = edge_index[0].astype(jnp.int32)
    col = edge_index[1].astype(jnp.int32)
    pad = epad - e
    rows_p = jnp.concatenate(
        [row, jnp.full((pad,), DUMMY, jnp.int32)]).reshape(NW, nb, EB)
    cols_p = jnp.concatenate(
        [col, jnp.zeros((pad,), jnp.int32)]).reshape(NW, nb, EB)
    onesd = jnp.ones((EB, D_FEAT), jnp.float32)
    zerosd = jnp.zeros((STRIPE, D_FEAT), jnp.float32)

    degp = _make_degree(nb)(rows_p, onesd, zerosd)

    prep = pl.pallas_call(
        _prep_body,
        out_shape=jax.ShapeDtypeStruct((N_NODES, D_FEAT), jnp.float32),
    )
    g0 = prep(x, degp)

    spmm = _make_spmm(nb)
    combine1 = pl.pallas_call(
        functools.partial(_combine_body, last_hop=False),
        out_shape=jax.ShapeDtypeStruct((N_NODES, D_FEAT), jnp.float32),
    )
    combine2 = pl.pallas_call(
        functools.partial(_combine_body, last_hop=True),
        out_shape=jax.ShapeDtypeStruct((N_NODES, D_FEAT), jnp.float32),
    )

    zp1 = spmm(g0, rows_p, cols_p, zerosd)
    g1 = combine1(zp1, g0, degp)
    zp2 = spmm(g1, rows_p, cols_p, zerosd)
    return combine2(zp2, g1, degp)
